# Initial kernel scaffold; baseline (speedup 1.0000x reference)
import jax
import jax.numpy as jnp
from jax.experimental import pallas as pl


def kernel(p_locs, p_conf, prior_boxes, targets):
    raise NotImplementedError("write your pallas kernel here")


# speedup vs baseline: 14.5188x; 14.5188x over previous
import functools

import jax
import jax.numpy as jnp
from jax import lax
from jax.experimental import pallas as pl
from jax.experimental.pallas import tpu as pltpu

_C = 21
_P = 8732
_G = 16
_B = 32
_ROWS = 72
_LANES = 128
_PP = _ROWS * _LANES
_VAR0 = 0.1
_VAR1 = 0.2
_THRESH = 0.5
_NEGPOS = 3


def _body(tgt_ref, plocs_ref, pconf_ref, priors_ref, ll_ref, lc_ref, np_ref):
    f32 = jnp.float32
    i32 = jnp.int32

    pcx = priors_ref[0]
    pcy = priors_ref[1]
    pw = priors_ref[2]
    ph = priors_ref[3]
    px0 = pcx - pw / 2.0
    py0 = pcy - ph / 2.0
    px1 = pcx + pw / 2.0
    py1 = pcy + ph / 2.0
    area_p = (px1 - px0) * (py1 - py0)

    p_lin = (lax.broadcasted_iota(i32, (_ROWS, _LANES), 0) * _LANES
             + lax.broadcasted_iota(i32, (_ROWS, _LANES), 1))
    pad = p_lin >= _P

    gts = []
    best = None
    bidx = None
    bpi = []
    for g in range(_G):
        b0 = 5 * g
        gx0 = tgt_ref[0, 0, b0 + 0]
        gy0 = tgt_ref[0, 0, b0 + 1]
        gx1 = tgt_ref[0, 0, b0 + 2]
        gy1 = tgt_ref[0, 0, b0 + 3]
        glab = tgt_ref[0, 0, b0 + 4]
        gts.append((gx0, gy0, gx1, gy1, glab))
        iw = jnp.clip(jnp.minimum(gx1, px1) - jnp.maximum(gx0, px0), 0.0, None)
        ih = jnp.clip(jnp.minimum(gy1, py1) - jnp.maximum(gy0, py0), 0.0, None)
        inter = iw * ih
        area_g = (gx1 - gx0) * (gy1 - gy0)
        iou = inter / (area_g + area_p - inter)
        iou = jnp.where(pad, -1.0, iou)
        mg = jnp.max(iou)
        bpi.append(jnp.min(jnp.where(iou == mg, p_lin, jnp.int32(2**30))))
        if g == 0:
            best = iou
            bidx = jnp.zeros((_ROWS, _LANES), i32)
        else:
            upd = iou > best
            best = jnp.where(upd, iou, best)
            bidx = jnp.where(upd, jnp.int32(g), bidx)

    for g in range(_G):
        claim = p_lin == bpi[g]
        best = jnp.where(claim, 2.0, best)
        bidx = jnp.where(claim, jnp.int32(g), bidx)

    m0 = jnp.zeros((_ROWS, _LANES), f32)
    m1 = jnp.zeros((_ROWS, _LANES), f32)
    m2 = jnp.zeros((_ROWS, _LANES), f32)
    m3 = jnp.zeros((_ROWS, _LANES), f32)
    lab = jnp.zeros((_ROWS, _LANES), f32)
    for g in range(_G):
        eq = bidx == g
        gx0, gy0, gx1, gy1, glab = gts[g]
        m0 = jnp.where(eq, gx0, m0)
        m1 = jnp.where(eq, gy0, m1)
        m2 = jnp.where(eq, gx1, m2)
        m3 = jnp.where(eq, gy1, m3)
        lab = jnp.where(eq, glab, lab)

    lab = jnp.where(best < _THRESH, 0.0, lab)
    pos = lab > 0.0
    posf = pos.astype(f32)
    num_pos = jnp.sum(posf)

    tx = ((m0 + m2) / 2.0 - pcx) / (_VAR0 * pw)
    ty = ((m1 + m3) / 2.0 - pcy) / (_VAR0 * ph)
    tw = jnp.log((m2 - m0) / pw) / _VAR1
    th = jnp.log((m3 - m1) / ph) / _VAR1
    acc = jnp.zeros((_ROWS, _LANES), f32)
    for j, t in enumerate((tx, ty, tw, th)):
        d = plocs_ref[0, j] - t
        ad = jnp.abs(d)
        sl1 = jnp.where(ad < 1.0, 0.5 * d * d, ad - 0.5)
        acc = acc + sl1 * posf
    loss_locs = jnp.sum(acc)

    m = pconf_ref[0, 0]
    for c in range(1, _C):
        m = jnp.maximum(m, pconf_ref[0, c])
    s = jnp.zeros((_ROWS, _LANES), f32)
    picked = jnp.zeros((_ROWS, _LANES), f32)
    for c in range(_C):
        xc = pconf_ref[0, c]
        s = s + jnp.exp(xc - m)
        picked = jnp.where(lab == float(c), xc, picked)
    ce = jnp.log(s) + m - picked

    lc_pos = jnp.sum(jnp.where(pos, ce, 0.0))

    temp = jnp.where(jnp.logical_or(pos, pad), 0.0, ce)
    ti = lax.bitcast_convert_type(temp, i32)
    k_f = jnp.minimum(_NEGPOS * num_pos, float(_P - 1))
    k_i = k_f.astype(i32)

    def bs_step(_, carry):
        lo, hi = carry
        mid = lo + ((hi - lo) >> 1)
        cnt = jnp.sum((ti >= mid).astype(i32))
        ok = cnt >= k_i
        return (jnp.where(ok, mid, lo), jnp.where(ok, hi, mid))

    lo, hi = lax.fori_loop(0, 31, bs_step,
                           (jnp.int32(0), jnp.int32(0x7F800000)))
    thr_i = lo
    thr_f = lax.bitcast_convert_type(thr_i, f32)
    gt_mask = ti > thr_i
    cnt_gt = jnp.sum(gt_mask.astype(i32)).astype(f32)
    sum_gt = jnp.sum(jnp.where(gt_mask, temp, 0.0))
    loss_conf = lc_pos + sum_gt + (k_f - cnt_gt) * thr_f

    ll_ref[0, 0, 0] = loss_locs
    lc_ref[0, 0, 0] = loss_conf
    np_ref[0, 0, 0] = num_pos


@jax.jit
def kernel(p_locs, p_conf, prior_boxes, targets):
    f32 = jnp.float32
    npad = _PP - _P
    pconf_t = jnp.transpose(p_conf, (0, 2, 1))
    pconf_t = jnp.pad(pconf_t, ((0, 0), (0, 0), (0, npad)))
    pconf_t = pconf_t.reshape(_B, _C, _ROWS, _LANES)
    plocs_t = jnp.transpose(p_locs, (0, 2, 1))
    plocs_t = jnp.pad(plocs_t, ((0, 0), (0, 0), (0, npad)))
    plocs_t = plocs_t.reshape(_B, 4, _ROWS, _LANES)
    pr = jnp.transpose(prior_boxes, (1, 0))
    padvals = jnp.broadcast_to(
        jnp.array([0.0, 0.0, 1.0, 1.0], f32)[:, None], (4, npad))
    pr = jnp.concatenate([pr, padvals], axis=1).reshape(4, _ROWS, _LANES)
    tgt = targets.reshape(_B, 1, _G * 5)

    grid = (_B,)
    ll, lc, npos = pl.pallas_call(
        _body,
        grid=grid,
        in_specs=[
            pl.BlockSpec((1, 1, _G * 5), lambda b: (b, 0, 0),
                         memory_space=pltpu.SMEM),
            pl.BlockSpec((1, 4, _ROWS, _LANES), lambda b: (b, 0, 0, 0)),
            pl.BlockSpec((1, _C, _ROWS, _LANES), lambda b: (b, 0, 0, 0)),
            pl.BlockSpec((4, _ROWS, _LANES), lambda b: (0, 0, 0)),
        ],
        out_specs=[
            pl.BlockSpec((1, 1, 1), lambda b: (b, 0, 0),
                         memory_space=pltpu.SMEM),
            pl.BlockSpec((1, 1, 1), lambda b: (b, 0, 0),
                         memory_space=pltpu.SMEM),
            pl.BlockSpec((1, 1, 1), lambda b: (b, 0, 0),
                         memory_space=pltpu.SMEM),
        ],
        out_shape=[
            jax.ShapeDtypeStruct((_B, 1, 1), f32),
            jax.ShapeDtypeStruct((_B, 1, 1), f32),
            jax.ShapeDtypeStruct((_B, 1, 1), f32),
        ],
        compiler_params=pltpu.CompilerParams(
            dimension_semantics=("parallel",)),
    )(tgt, plocs_t, pconf_t, pr)

    n = jnp.sum(npos)
    n = jnp.where(n == 0.0, 1.0, n)
    return jnp.sum(ll) / n, jnp.sum(lc) / n


# speedup vs baseline: 22.8322x; 1.5726x over previous
import functools

import jax
import jax.numpy as jnp
from jax import lax
from jax.experimental import pallas as pl
from jax.experimental.pallas import tpu as pltpu

_C = 21
_P = 8732
_G = 16
_B = 32
_ROWS = 72
_LANES = 128
_PP = _ROWS * _LANES
_VAR0 = 0.1
_VAR1 = 0.2
_THRESH = 0.5
_NEGPOS = 3


def _body(tgt_ref, plocs_ref, pconf_ref, priors_ref, temp_ref, ll_ref,
          lc_ref, np_ref):
    f32 = jnp.float32
    i32 = jnp.int32

    pcx = priors_ref[0]
    pcy = priors_ref[1]
    pw = priors_ref[2]
    ph = priors_ref[3]
    px0 = pcx - pw / 2.0
    py0 = pcy - ph / 2.0
    px1 = pcx + pw / 2.0
    py1 = pcy + ph / 2.0
    area_p = (px1 - px0) * (py1 - py0)

    p_lin = (lax.broadcasted_iota(i32, (_ROWS, _LANES), 0) * _LANES
             + lax.broadcasted_iota(i32, (_ROWS, _LANES), 1))
    pad = p_lin >= _P

    gts = []
    best = None
    bidx = None
    bpi = []
    for g in range(_G):
        b0 = 5 * g
        gx0 = tgt_ref[0, 0, b0 + 0]
        gy0 = tgt_ref[0, 0, b0 + 1]
        gx1 = tgt_ref[0, 0, b0 + 2]
        gy1 = tgt_ref[0, 0, b0 + 3]
        glab = tgt_ref[0, 0, b0 + 4]
        gts.append((gx0, gy0, gx1, gy1, glab))
        iw = jnp.clip(jnp.minimum(gx1, px1) - jnp.maximum(gx0, px0), 0.0, None)
        ih = jnp.clip(jnp.minimum(gy1, py1) - jnp.maximum(gy0, py0), 0.0, None)
        inter = iw * ih
        area_g = (gx1 - gx0) * (gy1 - gy0)
        iou = inter / (area_g + area_p - inter)
        iou = jnp.where(pad, -1.0, iou)
        mg = jnp.max(iou)
        bpi.append(jnp.min(jnp.where(iou == mg, p_lin, jnp.int32(2**30))))
        if g == 0:
            best = iou
            bidx = jnp.zeros((_ROWS, _LANES), i32)
        else:
            upd = iou > best
            best = jnp.where(upd, iou, best)
            bidx = jnp.where(upd, jnp.int32(g), bidx)

    for g in range(_G):
        claim = p_lin == bpi[g]
        best = jnp.where(claim, 2.0, best)
        bidx = jnp.where(claim, jnp.int32(g), bidx)

    m0 = jnp.zeros((_ROWS, _LANES), f32)
    m1 = jnp.zeros((_ROWS, _LANES), f32)
    m2 = jnp.zeros((_ROWS, _LANES), f32)
    m3 = jnp.zeros((_ROWS, _LANES), f32)
    lab = jnp.zeros((_ROWS, _LANES), f32)
    for g in range(_G):
        eq = bidx == g
        gx0, gy0, gx1, gy1, glab = gts[g]
        m0 = jnp.where(eq, gx0, m0)
        m1 = jnp.where(eq, gy0, m1)
        m2 = jnp.where(eq, gx1, m2)
        m3 = jnp.where(eq, gy1, m3)
        lab = jnp.where(eq, glab, lab)

    lab = jnp.where(best < _THRESH, 0.0, lab)
    pos = lab > 0.0
    posf = pos.astype(f32)
    num_pos = jnp.sum(posf)

    tx = ((m0 + m2) / 2.0 - pcx) / (_VAR0 * pw)
    ty = ((m1 + m3) / 2.0 - pcy) / (_VAR0 * ph)
    tw = jnp.log((m2 - m0) / pw) / _VAR1
    th = jnp.log((m3 - m1) / ph) / _VAR1
    acc = jnp.zeros((_ROWS, _LANES), f32)
    for j, t in enumerate((tx, ty, tw, th)):
        d = plocs_ref[0, j] - t
        ad = jnp.abs(d)
        sl1 = jnp.where(ad < 1.0, 0.5 * d * d, ad - 0.5)
        acc = acc + sl1 * posf
    loss_locs = jnp.sum(acc)

    m = pconf_ref[0, 0]
    for c in range(1, _C):
        m = jnp.maximum(m, pconf_ref[0, c])
    s = jnp.zeros((_ROWS, _LANES), f32)
    picked = jnp.zeros((_ROWS, _LANES), f32)
    for c in range(_C):
        xc = pconf_ref[0, c]
        s = s + jnp.exp(xc - m)
        picked = jnp.where(lab == float(c), xc, picked)
    ce = jnp.log(s) + m - picked

    lc_pos = jnp.sum(jnp.where(pos, ce, 0.0))

    temp_ref[0] = jnp.where(jnp.logical_or(pos, pad), 0.0, ce)

    ll_ref[0, 0, 0] = loss_locs
    lc_ref[0, 0, 0] = lc_pos
    np_ref[0, 0, 0] = num_pos


def _mine_body(temp_ref, ll_ref, lcpos_ref, npos_ref, oll_ref, olc_ref):
    f32 = jnp.float32
    i32 = jnp.int32
    temp = temp_ref[...]
    ti = lax.bitcast_convert_type(temp, i32)
    np_v = npos_ref[...]
    k_f = jnp.minimum(_NEGPOS * np_v, float(_P - 1))
    k_i = k_f.astype(i32)

    def bs_step(_, carry):
        lo, hi = carry
        mid = lo + ((hi - lo) >> 1)
        cnt = jnp.sum((ti >= mid).astype(i32), axis=1, keepdims=True)
        ok = cnt >= k_i
        return (jnp.where(ok, mid, lo), jnp.where(ok, hi, mid))

    init = (jnp.zeros((_B, 1), i32), jnp.full((_B, 1), 0x7F800000, i32))
    lo, _ = lax.fori_loop(0, 31, bs_step, init)
    thr_f = lax.bitcast_convert_type(lo, f32)
    gt_mask = ti > lo
    cnt_gt = jnp.sum(gt_mask.astype(f32), axis=1, keepdims=True)
    sum_gt = jnp.sum(jnp.where(gt_mask, temp, 0.0), axis=1, keepdims=True)
    lc_v = lcpos_ref[...] + sum_gt + (k_f - cnt_gt) * thr_f

    n = jnp.sum(np_v)
    n = jnp.where(n == 0.0, 1.0, n)
    oll_ref[0, 0] = jnp.sum(ll_ref[...]) / n
    olc_ref[0, 0] = jnp.sum(lc_v) / n


@jax.jit
def kernel(p_locs, p_conf, prior_boxes, targets):
    f32 = jnp.float32
    npad = _PP - _P
    pconf_t = jnp.transpose(p_conf, (0, 2, 1))
    pconf_t = jnp.pad(pconf_t, ((0, 0), (0, 0), (0, npad)))
    pconf_t = pconf_t.reshape(_B, _C, _ROWS, _LANES)
    plocs_t = jnp.transpose(p_locs, (0, 2, 1))
    plocs_t = jnp.pad(plocs_t, ((0, 0), (0, 0), (0, npad)))
    plocs_t = plocs_t.reshape(_B, 4, _ROWS, _LANES)
    pr = jnp.transpose(prior_boxes, (1, 0))
    padvals = jnp.broadcast_to(
        jnp.array([0.0, 0.0, 1.0, 1.0], f32)[:, None], (4, npad))
    pr = jnp.concatenate([pr, padvals], axis=1).reshape(4, _ROWS, _LANES)
    tgt = targets.reshape(_B, 1, _G * 5)

    grid = (_B,)
    temp, ll, lcpos, npos = pl.pallas_call(
        _body,
        grid=grid,
        in_specs=[
            pl.BlockSpec((1, 1, _G * 5), lambda b: (b, 0, 0),
                         memory_space=pltpu.SMEM),
            pl.BlockSpec((1, 4, _ROWS, _LANES), lambda b: (b, 0, 0, 0)),
            pl.BlockSpec((1, _C, _ROWS, _LANES), lambda b: (b, 0, 0, 0)),
            pl.BlockSpec((4, _ROWS, _LANES), lambda b: (0, 0, 0)),
        ],
        out_specs=[
            pl.BlockSpec((1, _ROWS, _LANES), lambda b: (b, 0, 0)),
            pl.BlockSpec((1, 1, 1), lambda b: (b, 0, 0),
                         memory_space=pltpu.SMEM),
            pl.BlockSpec((1, 1, 1), lambda b: (b, 0, 0),
                         memory_space=pltpu.SMEM),
            pl.BlockSpec((1, 1, 1), lambda b: (b, 0, 0),
                         memory_space=pltpu.SMEM),
        ],
        out_shape=[
            jax.ShapeDtypeStruct((_B, _ROWS, _LANES), f32),
            jax.ShapeDtypeStruct((_B, 1, 1), f32),
            jax.ShapeDtypeStruct((_B, 1, 1), f32),
            jax.ShapeDtypeStruct((_B, 1, 1), f32),
        ],
        compiler_params=pltpu.CompilerParams(
            dimension_semantics=("parallel",)),
    )(tgt, plocs_t, pconf_t, pr)

    oll, olc = pl.pallas_call(
        _mine_body,
        in_specs=[
            pl.BlockSpec((_B, _PP), lambda: (0, 0)),
            pl.BlockSpec((_B, 1), lambda: (0, 0)),
            pl.BlockSpec((_B, 1), lambda: (0, 0)),
            pl.BlockSpec((_B, 1), lambda: (0, 0)),
        ],
        out_specs=[
            pl.BlockSpec((1, 1), lambda: (0, 0), memory_space=pltpu.SMEM),
            pl.BlockSpec((1, 1), lambda: (0, 0), memory_space=pltpu.SMEM),
        ],
        out_shape=[
            jax.ShapeDtypeStruct((1, 1), f32),
            jax.ShapeDtypeStruct((1, 1), f32),
        ],
    )(temp.reshape(_B, _PP), ll.reshape(_B, 1), lcpos.reshape(_B, 1),
      npos.reshape(_B, 1))

    return oll[0, 0], olc[0, 0]


# speedup vs baseline: 23.1412x; 1.0135x over previous
import functools

import jax
import jax.numpy as jnp
from jax import lax
from jax.experimental import pallas as pl
from jax.experimental.pallas import tpu as pltpu

_C = 21
_P = 8732
_G = 16
_B = 32
_ROWS = 72
_LANES = 128
_PP = _ROWS * _LANES
_BPB = 2
_VAR0 = 0.1
_VAR1 = 0.2
_THRESH = 0.5
_NEGPOS = 3


def _body(tgt_ref, plocs_ref, pconf_ref, priors_ref, temp_ref, ll_ref,
          lc_ref, np_ref):
    for i in range(_BPB):
        _one_batch(i, tgt_ref, plocs_ref, pconf_ref, priors_ref, temp_ref,
                   ll_ref, lc_ref, np_ref)


def _one_batch(i, tgt_ref, plocs_ref, pconf_ref, priors_ref, temp_ref,
               ll_ref, lc_ref, np_ref):
    f32 = jnp.float32
    i32 = jnp.int32

    pcx = priors_ref[0]
    pcy = priors_ref[1]
    pw = priors_ref[2]
    ph = priors_ref[3]
    px0 = pcx - pw / 2.0
    py0 = pcy - ph / 2.0
    px1 = pcx + pw / 2.0
    py1 = pcy + ph / 2.0
    area_p = (px1 - px0) * (py1 - py0)

    p_lin = (lax.broadcasted_iota(i32, (_ROWS, _LANES), 0) * _LANES
             + lax.broadcasted_iota(i32, (_ROWS, _LANES), 1))
    pad = p_lin >= _P

    gts = []
    best = None
    bidx = None
    bpi = []
    for g in range(_G):
        b0 = 5 * g
        gx0 = tgt_ref[i, 0, b0 + 0]
        gy0 = tgt_ref[i, 0, b0 + 1]
        gx1 = tgt_ref[i, 0, b0 + 2]
        gy1 = tgt_ref[i, 0, b0 + 3]
        glab = tgt_ref[i, 0, b0 + 4]
        gts.append((gx0, gy0, gx1, gy1, glab))
        iw = jnp.clip(jnp.minimum(gx1, px1) - jnp.maximum(gx0, px0), 0.0, None)
        ih = jnp.clip(jnp.minimum(gy1, py1) - jnp.maximum(gy0, py0), 0.0, None)
        inter = iw * ih
        area_g = (gx1 - gx0) * (gy1 - gy0)
        iou = inter / (area_g + area_p - inter)
        iou = jnp.where(pad, -1.0, iou)
        mg = jnp.max(iou)
        bpi.append(jnp.min(jnp.where(iou == mg, p_lin, jnp.int32(2**30))))
        if g == 0:
            best = iou
            bidx = jnp.zeros((_ROWS, _LANES), i32)
        else:
            upd = iou > best
            best = jnp.where(upd, iou, best)
            bidx = jnp.where(upd, jnp.int32(g), bidx)

    for g in range(_G):
        claim = p_lin == bpi[g]
        best = jnp.where(claim, 2.0, best)
        bidx = jnp.where(claim, jnp.int32(g), bidx)

    m0 = jnp.zeros((_ROWS, _LANES), f32)
    m1 = jnp.zeros((_ROWS, _LANES), f32)
    m2 = jnp.zeros((_ROWS, _LANES), f32)
    m3 = jnp.zeros((_ROWS, _LANES), f32)
    lab = jnp.zeros((_ROWS, _LANES), f32)
    for g in range(_G):
        eq = bidx == g
        gx0, gy0, gx1, gy1, glab = gts[g]
        m0 = jnp.where(eq, gx0, m0)
        m1 = jnp.where(eq, gy0, m1)
        m2 = jnp.where(eq, gx1, m2)
        m3 = jnp.where(eq, gy1, m3)
        lab = jnp.where(eq, glab, lab)

    lab = jnp.where(best < _THRESH, 0.0, lab)
    pos = lab > 0.0
    posf = pos.astype(f32)
    num_pos = jnp.sum(posf)

    tx = ((m0 + m2) / 2.0 - pcx) / (_VAR0 * pw)
    ty = ((m1 + m3) / 2.0 - pcy) / (_VAR0 * ph)
    tw = jnp.log((m2 - m0) / pw) / _VAR1
    th = jnp.log((m3 - m1) / ph) / _VAR1
    acc = jnp.zeros((_ROWS, _LANES), f32)
    for j, t in enumerate((tx, ty, tw, th)):
        d = plocs_ref[i, j] - t
        ad = jnp.abs(d)
        sl1 = jnp.where(ad < 1.0, 0.5 * d * d, ad - 0.5)
        acc = acc + sl1 * posf
    loss_locs = jnp.sum(acc)

    m = pconf_ref[i, 0]
    for c in range(1, _C):
        m = jnp.maximum(m, pconf_ref[i, c])
    s = jnp.zeros((_ROWS, _LANES), f32)
    picked = jnp.zeros((_ROWS, _LANES), f32)
    for c in range(_C):
        xc = pconf_ref[i, c]
        s = s + jnp.exp(xc - m)
        picked = jnp.where(lab == float(c), xc, picked)
    ce = jnp.log(s) + m - picked

    lc_pos = jnp.sum(jnp.where(pos, ce, 0.0))

    temp_ref[i] = jnp.where(jnp.logical_or(pos, pad), 0.0, ce)

    ll_ref[i, 0, 0] = loss_locs
    lc_ref[i, 0, 0] = lc_pos
    np_ref[i, 0, 0] = num_pos


def _mine_body(temp_ref, ll_ref, lcpos_ref, npos_ref, oll_ref, olc_ref):
    f32 = jnp.float32
    i32 = jnp.int32
    temp = temp_ref[...]
    ti = lax.bitcast_convert_type(temp, i32)
    np_v = npos_ref[...]
    k_f = jnp.minimum(_NEGPOS * np_v, float(_P - 1))
    k_i = k_f.astype(i32)

    def bs_step(_, carry):
        lo, hi = carry
        mid = lo + ((hi - lo) >> 1)
        cnt = jnp.sum((ti >= mid).astype(i32), axis=1, keepdims=True)
        ok = cnt >= k_i
        return (jnp.where(ok, mid, lo), jnp.where(ok, hi, mid))

    init = (jnp.zeros((_B, 1), i32), jnp.full((_B, 1), 0x7F800000, i32))
    lo, _ = lax.fori_loop(0, 31, bs_step, init)
    thr_f = lax.bitcast_convert_type(lo, f32)
    gt_mask = ti > lo
    cnt_gt = jnp.sum(gt_mask.astype(f32), axis=1, keepdims=True)
    sum_gt = jnp.sum(jnp.where(gt_mask, temp, 0.0), axis=1, keepdims=True)
    lc_v = lcpos_ref[...] + sum_gt + (k_f - cnt_gt) * thr_f

    n = jnp.sum(np_v)
    n = jnp.where(n == 0.0, 1.0, n)
    oll_ref[0, 0] = jnp.sum(ll_ref[...]) / n
    olc_ref[0, 0] = jnp.sum(lc_v) / n


@jax.jit
def kernel(p_locs, p_conf, prior_boxes, targets):
    f32 = jnp.float32
    npad = _PP - _P
    pconf_t = jnp.transpose(p_conf, (0, 2, 1))
    pconf_t = jnp.pad(pconf_t, ((0, 0), (0, 0), (0, npad)))
    pconf_t = pconf_t.reshape(_B, _C, _ROWS, _LANES)
    plocs_t = jnp.transpose(p_locs, (0, 2, 1))
    plocs_t = jnp.pad(plocs_t, ((0, 0), (0, 0), (0, npad)))
    plocs_t = plocs_t.reshape(_B, 4, _ROWS, _LANES)
    pr = jnp.transpose(prior_boxes, (1, 0))
    padvals = jnp.broadcast_to(
        jnp.array([0.0, 0.0, 1.0, 1.0], f32)[:, None], (4, npad))
    pr = jnp.concatenate([pr, padvals], axis=1).reshape(4, _ROWS, _LANES)
    tgt = targets.reshape(_B, 1, _G * 5)

    grid = (_B // _BPB,)
    temp, ll, lcpos, npos = pl.pallas_call(
        _body,
        grid=grid,
        in_specs=[
            pl.BlockSpec((_BPB, 1, _G * 5), lambda b: (b, 0, 0),
                         memory_space=pltpu.SMEM),
            pl.BlockSpec((_BPB, 4, _ROWS, _LANES), lambda b: (b, 0, 0, 0)),
            pl.BlockSpec((_BPB, _C, _ROWS, _LANES), lambda b: (b, 0, 0, 0)),
            pl.BlockSpec((4, _ROWS, _LANES), lambda b: (0, 0, 0)),
        ],
        out_specs=[
            pl.BlockSpec((_BPB, _ROWS, _LANES), lambda b: (b, 0, 0)),
            pl.BlockSpec((_BPB, 1, 1), lambda b: (b, 0, 0),
                         memory_space=pltpu.SMEM),
            pl.BlockSpec((_BPB, 1, 1), lambda b: (b, 0, 0),
                         memory_space=pltpu.SMEM),
            pl.BlockSpec((_BPB, 1, 1), lambda b: (b, 0, 0),
                         memory_space=pltpu.SMEM),
        ],
        out_shape=[
            jax.ShapeDtypeStruct((_B, _ROWS, _LANES), f32),
            jax.ShapeDtypeStruct((_B, 1, 1), f32),
            jax.ShapeDtypeStruct((_B, 1, 1), f32),
            jax.ShapeDtypeStruct((_B, 1, 1), f32),
        ],
    )(tgt, plocs_t, pconf_t, pr)

    oll, olc = pl.pallas_call(
        _mine_body,
        in_specs=[
            pl.BlockSpec((_B, _PP), lambda: (0, 0)),
            pl.BlockSpec((_B, 1), lambda: (0, 0)),
            pl.BlockSpec((_B, 1), lambda: (0, 0)),
            pl.BlockSpec((_B, 1), lambda: (0, 0)),
        ],
        out_specs=[
            pl.BlockSpec((1, 1), lambda: (0, 0), memory_space=pltpu.SMEM),
            pl.BlockSpec((1, 1), lambda: (0, 0), memory_space=pltpu.SMEM),
        ],
        out_shape=[
            jax.ShapeDtypeStruct((1, 1), f32),
            jax.ShapeDtypeStruct((1, 1), f32),
        ],
    )(temp.reshape(_B, _PP), ll.reshape(_B, 1), lcpos.reshape(_B, 1),
      npos.reshape(_B, 1))

    return oll[0, 0], olc[0, 0]
